# baseline (speedup 1.0000x reference)
import functools

import jax
import jax.numpy as jnp
from jax import lax
from jax.experimental import pallas as pl
from jax.experimental.pallas import tpu as pltpu
from jax.experimental.pallas import tpu_sc as plsc

N = 10000
E = 320000
G = 64
DIN = 128
DHID = 256
NP = 10240
NTILE = 16
ROWS_PER_TILE = NP // NTILE
CHUNK = 128
CH_AGG = 160
GRP = 16
NGRP = CH_AGG // GRP
EP = NTILE * CH_AGG * CHUNK
TPE_DEG = EP // 32
RB = 1000
NB = N // RB

_INV_SQRT2 = 0.7071067811865476


def _erf(x):
    s = jnp.sign(x)
    a = jnp.abs(x)
    t = 1.0 / (1.0 + 0.3275911 * a)
    poly = t * (0.254829592 + t * (-0.284496736 + t * (1.421413741
               + t * (-1.453152027 + t * 1.061405429))))
    return s * (1.0 - poly * jnp.exp(-a * a))


def _gelu(v):
    return v * 0.5 * (1.0 + _erf(v * _INV_SQRT2))


def _deg_call(dst3, ones):
    mesh = plsc.VectorSubcoreMesh(core_axis_name="c", subcore_axis_name="s")

    @functools.partial(
        pl.kernel, mesh=mesh,
        out_type=jax.ShapeDtypeStruct((2, NP, 128), jnp.float32),
        scratch_types=[
            pltpu.VMEM((GRP, CHUNK), jnp.int32),
            pltpu.VMEM((CHUNK, 128), jnp.float32),
            pltpu.VMEM((16, 128), jnp.float32),
            pltpu.VMEM_SHARED((NP, 128), jnp.float32),
        ],
    )
    def k(dst_hbm, ones_hbm, out_hbm, dst_v, ones_v, zb_v, acc_s):
        c = lax.axis_index("c")
        t = lax.axis_index("s")
        zero16 = jnp.zeros((16,), jnp.float32)
        for r in range(16):
            for cc in range(8):
                zb_v[r, pl.ds(cc * 16, 16)] = zero16

        def zslab(j, _):
            pltpu.sync_copy(zb_v, acc_s.at[pl.ds(t * ROWS_PER_TILE + j * 16, 16)])
            return 0
        lax.fori_loop(0, ROWS_PER_TILE // 16, zslab, 0)

        pltpu.sync_copy(ones_hbm, ones_v)
        plsc.subcore_barrier()

        def grp(g, _):
            pltpu.sync_copy(dst_hbm.at[t].at[pl.ds(g * GRP, GRP)], dst_v)

            def body(j, _):
                pltpu.sync_copy(ones_v, acc_s.at[dst_v.at[j]], add=True)
                return 0
            lax.fori_loop(0, GRP, body, 0)
            return 0
        lo = c * (NGRP // 2)
        lax.fori_loop(lo, lo + NGRP // 2, grp, 0)

        plsc.subcore_barrier()
        sl = pl.ds(t * ROWS_PER_TILE, ROWS_PER_TILE)
        pltpu.sync_copy(acc_s.at[sl], out_hbm.at[c].at[sl])

    return k(dst3, ones)


def _agg_call(h, src3, dst3, split_cols):
    mesh = plsc.VectorSubcoreMesh(core_axis_name="c", subcore_axis_name="s")

    @functools.partial(
        pl.kernel, mesh=mesh,
        out_type=jax.ShapeDtypeStruct((2, NP, 128), jnp.float32),
        scratch_types=[
            pltpu.VMEM((GRP, CHUNK), jnp.int32),
            pltpu.VMEM((GRP, CHUNK), jnp.int32),
            pltpu.VMEM((CHUNK, 128), jnp.float32),
            pltpu.VMEM((CHUNK, 128), jnp.float32),
            pltpu.VMEM((16, 128), jnp.float32),
            pltpu.VMEM_SHARED((NP, 128), jnp.float32),
            pltpu.SemaphoreType.DMA,
            pltpu.SemaphoreType.DMA,
        ],
    )
    def k(h_hbm, src_hbm, dst_hbm, out_hbm, src_v, dst_v, rows_a, rows_b,
          zb_v, acc_s, sem_a, sem_b):
        c = lax.axis_index("c")
        t = lax.axis_index("s")
        zero16 = jnp.zeros((16,), jnp.float32)
        for r in range(16):
            for cc in range(8):
                zb_v[r, pl.ds(cc * 16, 16)] = zero16

        def zslab(j, _):
            pltpu.sync_copy(zb_v, acc_s.at[pl.ds(t * ROWS_PER_TILE + j * 16, 16)])
            return 0
        lax.fori_loop(0, ROWS_PER_TILE // 16, zslab, 0)

        plsc.subcore_barrier()

        if split_cols:
            hview = h_hbm.at[c]
        else:
            hview = h_hbm
        dummy = hview.at[pl.ds(0, CHUNK)]

        def gath(j, buf, sem):
            pltpu.async_copy(hview.at[src_v.at[j]], buf, sem)

        def wait(buf, sem):
            pltpu.make_async_copy(dummy, buf, sem).wait()

        def grp(g, _):
            pltpu.sync_copy(src_hbm.at[t].at[pl.ds(g * GRP, GRP)], src_v)
            pltpu.sync_copy(dst_hbm.at[t].at[pl.ds(g * GRP, GRP)], dst_v)
            gath(0, rows_a, sem_a)

            def pair(p, _):
                j0 = 2 * p
                wait(rows_a, sem_a)
                gath(j0 + 1, rows_b, sem_b)
                pltpu.sync_copy(rows_a, acc_s.at[dst_v.at[j0]], add=True)
                wait(rows_b, sem_b)

                @pl.when(p < GRP // 2 - 1)
                def _():
                    gath(j0 + 2, rows_a, sem_a)

                pltpu.sync_copy(rows_b, acc_s.at[dst_v.at[j0 + 1]], add=True)
                return 0
            lax.fori_loop(0, GRP // 2, pair, 0)
            return 0
        if split_cols:
            lax.fori_loop(0, NGRP, grp, 0)
        else:
            lo = c * (NGRP // 2)
            lax.fori_loop(lo, lo + NGRP // 2, grp, 0)

        plsc.subcore_barrier()
        sl = pl.ds(t * ROWS_PER_TILE, ROWS_PER_TILE)
        pltpu.sync_copy(acc_s.at[sl], out_hbm.at[c].at[sl])

    return k(h, src3, dst3)


def _prep_call(deg2, x, W0):
    def body(deg_ref, x_ref, w_ref, h1p_ref, h1f_ref, dinv_ref):
        deg = deg_ref[0, :, 0:1] + deg_ref[1, :, 0:1] + 1.0
        dinv = 1.0 / jnp.sqrt(deg)
        dinv_ref[...] = jnp.broadcast_to(dinv, (RB, 8))
        h1 = jnp.dot(x_ref[...], w_ref[...], preferred_element_type=jnp.float32,
                     precision=lax.Precision.DEFAULT)
        h1f_ref[...] = h1
        h1s = h1 * dinv
        h1p_ref[0] = h1s[:, : DHID // 2]
        h1p_ref[1] = h1s[:, DHID // 2:]

    return pl.pallas_call(
        body,
        grid=(NB,),
        in_specs=[
            pl.BlockSpec((2, RB, 128), lambda r: (0, r, 0)),
            pl.BlockSpec((RB, DIN), lambda r: (r, 0)),
            pl.BlockSpec((DIN, DHID), lambda r: (0, 0)),
        ],
        out_specs=[
            pl.BlockSpec((2, RB, DHID // 2), lambda r: (0, r, 0)),
            pl.BlockSpec((RB, DHID), lambda r: (r, 0)),
            pl.BlockSpec((RB, 8), lambda r: (r, 0)),
        ],
        out_shape=[
            jax.ShapeDtypeStruct((2, NP, DHID // 2), jnp.float32),
            jax.ShapeDtypeStruct((N, DHID), jnp.float32),
            jax.ShapeDtypeStruct((NP, 8), jnp.float32),
        ],
    )(deg2, x, W0)


def _conv_out(agg_ref, h1_ref, dinv_ref, b_ref):
    dinv_col = dinv_ref[:, 0:1]
    dd = dinv_col * dinv_col
    aggC = jnp.concatenate([agg_ref[0], agg_ref[1]], axis=1)
    return (aggC * dinv_col + h1_ref[...] * dd) + b_ref[...]


def _post_call(agg, h1f, dinv, b):
    def body(agg_ref, h1_ref, dinv_ref, b_ref, st_ref):
        r = pl.program_id(0)
        conv = _conv_out(agg_ref, h1_ref, dinv_ref, b_ref)

        @pl.when(r == 0)
        def _():
            st_ref[...] = jnp.zeros_like(st_ref)

        st_ref[0:1, :] += jnp.sum(conv, axis=0, keepdims=True)
        st_ref[1:2, :] += jnp.sum(conv * conv, axis=0, keepdims=True)

    return pl.pallas_call(
        body,
        grid=(NB,),
        in_specs=[
            pl.BlockSpec((2, RB, DHID // 2), lambda r: (0, r, 0)),
            pl.BlockSpec((RB, DHID), lambda r: (r, 0)),
            pl.BlockSpec((RB, 8), lambda r: (r, 0)),
            pl.BlockSpec((1, DHID), lambda r: (0, 0)),
        ],
        out_specs=pl.BlockSpec((8, DHID), lambda r: (0, 0)),
        out_shape=jax.ShapeDtypeStruct((8, DHID), jnp.float32),
    )(agg, h1f, dinv, b)


def _bngelu_dense_call(agg, h1f, dinv, b, st, g, be, Wn):
    def body(agg_ref, h1_ref, dinv_ref, b_ref, st_ref, g_ref, be_ref, w_ref,
             h1p_ref, h1f_ref):
        conv = _conv_out(agg_ref, h1_ref, dinv_ref, b_ref)
        mu = st_ref[0:1, :] / N
        ex2 = st_ref[1:2, :] / N
        Hn = (conv - mu) / jnp.sqrt(ex2 - mu * mu + 1e-5) * g_ref[...] \
            + be_ref[...]
        H = _gelu(Hn)
        h1 = jnp.dot(H, w_ref[...], preferred_element_type=jnp.float32,
                     precision=lax.Precision.DEFAULT)
        h1f_ref[...] = h1
        h1s = h1 * dinv_ref[:, 0:1]
        h1p_ref[0] = h1s[:, : DHID // 2]
        h1p_ref[1] = h1s[:, DHID // 2:]

    return pl.pallas_call(
        body,
        grid=(NB,),
        in_specs=[
            pl.BlockSpec((2, RB, DHID // 2), lambda r: (0, r, 0)),
            pl.BlockSpec((RB, DHID), lambda r: (r, 0)),
            pl.BlockSpec((RB, 8), lambda r: (r, 0)),
            pl.BlockSpec((1, DHID), lambda r: (0, 0)),
            pl.BlockSpec((8, DHID), lambda r: (0, 0)),
            pl.BlockSpec((1, DHID), lambda r: (0, 0)),
            pl.BlockSpec((1, DHID), lambda r: (0, 0)),
            pl.BlockSpec((DHID, DHID), lambda r: (0, 0)),
        ],
        out_specs=[
            pl.BlockSpec((2, RB, DHID // 2), lambda r: (0, r, 0)),
            pl.BlockSpec((RB, DHID), lambda r: (r, 0)),
        ],
        out_shape=[
            jax.ShapeDtypeStruct((2, NP, DHID // 2), jnp.float32),
            jax.ShapeDtypeStruct((N, DHID), jnp.float32),
        ],
    )(agg, h1f, dinv, b, st, g, be, Wn)


def _tail_call(agg2, h1f2, dinv, b2, batch3, Wh1, bh1, Wh2, bh2, Wo, bo):
    def body(agg_ref, h1_ref, dinv_ref, b2_ref, bt_ref,
             wh1_ref, bh1_ref, wh2_ref, bh2_ref, wo_ref, bo_ref,
             out_ref, sums_ref, cnt_ref):
        r = pl.program_id(0)
        Z = _conv_out(agg_ref, h1_ref, dinv_ref, b2_ref)
        bvec = bt_ref[0, 0, :]
        gid = lax.broadcasted_iota(jnp.int32, (G, RB), 0)
        M = (gid == bvec[None, :]).astype(jnp.float32)

        @pl.when(r == 0)
        def _():
            sums_ref[...] = jnp.zeros_like(sums_ref)
            cnt_ref[...] = jnp.zeros_like(cnt_ref)

        sums_ref[...] += jnp.dot(M, Z, preferred_element_type=jnp.float32,
                                 precision=lax.Precision.HIGHEST)
        cnt_ref[...] += jnp.broadcast_to(
            jnp.sum(M, axis=1, keepdims=True), (G, 128))

        @pl.when(r == NB - 1)
        def _():
            pooled = sums_ref[...] / jnp.maximum(cnt_ref[:, 0:1], 1.0)
            z1 = _gelu(jnp.dot(pooled, wh1_ref[...],
                               preferred_element_type=jnp.float32,
                               precision=lax.Precision.DEFAULT) + bh1_ref[...])
            z2 = _gelu(jnp.dot(z1, wh2_ref[...],
                               preferred_element_type=jnp.float32,
                               precision=lax.Precision.DEFAULT) + bh2_ref[...])
            out_ref[...] = jnp.dot(z2, wo_ref[...],
                                   preferred_element_type=jnp.float32,
                                   precision=lax.Precision.DEFAULT) \
                + bo_ref[0:1, 0:1]

    return pl.pallas_call(
        body,
        grid=(NB,),
        in_specs=[
            pl.BlockSpec((2, RB, DHID // 2), lambda r: (0, r, 0)),
            pl.BlockSpec((RB, DHID), lambda r: (r, 0)),
            pl.BlockSpec((RB, 8), lambda r: (r, 0)),
            pl.BlockSpec((1, DHID), lambda r: (0, 0)),
            pl.BlockSpec((1, 1, RB), lambda r: (r, 0, 0)),
            pl.BlockSpec((DHID, DHID), lambda r: (0, 0)),
            pl.BlockSpec((1, DHID), lambda r: (0, 0)),
            pl.BlockSpec((DHID, DHID), lambda r: (0, 0)),
            pl.BlockSpec((1, DHID), lambda r: (0, 0)),
            pl.BlockSpec((DHID, 1), lambda r: (0, 0)),
            pl.BlockSpec((8, 128), lambda r: (0, 0)),
        ],
        out_specs=pl.BlockSpec((G, 1), lambda r: (0, 0)),
        out_shape=jax.ShapeDtypeStruct((G, 1), jnp.float32),
        scratch_shapes=[
            pltpu.VMEM((G, DHID), jnp.float32),
            pltpu.VMEM((G, 128), jnp.float32),
        ],
    )(agg2, h1f2, dinv, b2, batch3, Wh1, bh1, Wh2, bh2, Wo, bo)


def kernel(x, edge_index, batch, W0, b0, W1, b1, W2, b2, g0, be0, g1, be1,
           Wh1, bh1, Wh2, bh2, Wo, bo):
    src = edge_index[0]
    dst = edge_index[1]
    pad = N + (jnp.arange(EP - E, dtype=jnp.int32) % (NP - N))
    src_p = jnp.concatenate([src, pad])
    dst_p = jnp.concatenate([dst, pad])
    src3 = src_p.reshape(NTILE, CH_AGG, CHUNK)
    dst3 = dst_p.reshape(NTILE, CH_AGG, CHUNK)
    batch3 = batch.reshape(NB, 1, RB)

    ones = jnp.ones((CHUNK, 128), jnp.float32)
    deg2 = _deg_call(dst3, ones)
    h1p, h1f, dinv = _prep_call(deg2, x, W0)

    agg0 = _agg_call(h1p, src3, dst3, split_cols=True)
    st0 = _post_call(agg0, h1f, dinv, b0.reshape(1, DHID))
    h1p, h1f = _bngelu_dense_call(agg0, h1f, dinv, b0.reshape(1, DHID), st0,
                                  g0.reshape(1, DHID), be0.reshape(1, DHID), W1)

    agg1 = _agg_call(h1p, src3, dst3, split_cols=True)
    st1 = _post_call(agg1, h1f, dinv, b1.reshape(1, DHID))
    h1p, h1f = _bngelu_dense_call(agg1, h1f, dinv, b1.reshape(1, DHID), st1,
                                  g1.reshape(1, DHID), be1.reshape(1, DHID), W2)

    agg2 = _agg_call(h1p, src3, dst3, split_cols=True)
    return _tail_call(agg2, h1f, dinv, b2.reshape(1, DHID), batch3,
                      Wh1, bh1.reshape(1, DHID), Wh2, bh2.reshape(1, DHID),
                      Wo, jnp.broadcast_to(bo.reshape(1, 1), (8, 128)))
